# baseline (speedup 1.0000x reference)
import jax
import jax.numpy as jnp
from jax import lax
from jax.experimental import pallas as pl
from jax.experimental.pallas import tpu as pltpu
from jax.experimental.pallas import tpu_sc as plsc

N = 10000
E = 320000
D = 128
H = 128
HP = 144
C = 10
G = 64

NC = 2
NS = 16
NW = NC * NS
CH = 128
NCH0 = 106
NCH1 = 54
NCHT = NS * (NCH0 + NCH1)
RPT = N // NS

_SC_PARAMS = pltpu.CompilerParams(needs_layout_passes=False,
                                  use_tc_tiling_on_sc=False)


def _tile_plan(c, s):
  cbase = jnp.where(c == 0, s * NCH0, NS * NCH0 + s * NCH1)
  ncz = jnp.where(c == 0, NCH0, NCH1)
  return cbase, ncz


def _sc_pw_kernel(aa_hbm, src_hbm, dst_hbm, p_hbm,
                  asrc_v, adst_v, src2_v, dst2_v, p2_v):
  c = lax.axis_index("c")
  s = lax.axis_index("s")
  tid = c * NS + s
  cbase, ncz = _tile_plan(c, s)

  pltpu.sync_copy(aa_hbm.at[0], asrc_v)
  pltpu.sync_copy(aa_hbm.at[1], adst_v)
  pltpu.sync_copy(src_hbm.at[tid], src2_v)
  pltpu.sync_copy(dst_hbm.at[tid], dst2_v)

  @pl.loop(0, ncz)
  def _chunk(j):
    for i in range(CH // 16):
      s16 = src2_v[j, pl.ds(i * 16, 16)]
      d16 = dst2_v[j, pl.ds(i * 16, 16)]
      av = plsc.load_gather(asrc_v, [s16])
      dv = plsc.load_gather(adst_v, [d16])
      e = av + dv
      e = jnp.where(e < 0.0, 0.2 * e, e)
      p = jnp.exp(e)
      eid = (cbase + j) * CH + i * 16 + lax.iota(jnp.int32, 16)
      p = jnp.where(eid < E, p, 0.0)
      p2_v[j, pl.ds(i * 16, 16)] = p

  pltpu.sync_copy(p2_v, p_hbm.at[tid])


def _make_sc_pw():
  mesh = plsc.VectorSubcoreMesh(core_axis_name="c", subcore_axis_name="s")
  return pl.kernel(
      _sc_pw_kernel,
      out_type=jax.ShapeDtypeStruct((NW, NCH0, CH), jnp.float32),
      mesh=mesh,
      compiler_params=_SC_PARAMS,
      scratch_types=[
          pltpu.VMEM((N,), jnp.float32),
          pltpu.VMEM((N,), jnp.float32),
          pltpu.VMEM((NCH0, CH), jnp.int32),
          pltpu.VMEM((NCH0, CH), jnp.int32),
          pltpu.VMEM((NCH0, CH), jnp.float32),
      ],
  )


def _sc_rows_kernel(h_hbm, sdp_hbm,
                    acc_hbm,
                    sdp_v, rows_v, acc_sh,
                    sem_g0, sem_g1, sem_s0, sem_s1):
  c = lax.axis_index("c")
  s = lax.axis_index("s")
  cbase, ncz = _tile_plan(c, s)
  sem_g = (sem_g0, sem_g1)
  sem_s = (sem_s0, sem_s1)

  def stage(b, j):
    pltpu.sync_copy(sdp_hbm.at[cbase + j], sdp_v.at[b])

  def issue_gather(b):
    return pltpu.async_copy(h_hbm.at[sdp_v.at[b, 0]], rows_v.at[b],
                            sem_g[b])

  def wait_gather(b):
    pltpu.make_async_copy(h_hbm.at[sdp_v.at[b, 0]], rows_v.at[b],
                          sem_g[b]).wait()

  def issue_scatter(b):
    return pltpu.async_copy(rows_v.at[b], acc_sh.at[sdp_v.at[b, 1]],
                            sem_s[b], add=True)

  def wait_scatter(b):
    pltpu.make_async_copy(rows_v.at[b], acc_sh.at[sdp_v.at[b, 1]],
                          sem_s[b]).wait()

  def scale(b):
    for k in range(CH // 16):
      p16 = plsc.bitcast(sdp_v[b, 2, pl.ds(k * 16, 16)], jnp.float32)
      for t in range(16):
        r = k * 16 + t
        pr = p16[t]
        for i in range(HP // 16):
          rows_v[b, r, pl.ds(i * 16, 16)] = (
              rows_v[b, r, pl.ds(i * 16, 16)] * pr)

  with jax.named_scope("acc_zero"):
    @pl.loop(0, CH)
    def _zero_rows(r):
      for i in range(HP // 16):
        rows_v[0, r, pl.ds(i * 16, 16)] = jnp.zeros((16,), jnp.float32)

    for k in range(RPT // CH):
      pltpu.sync_copy(rows_v.at[0],
                      acc_sh.at[pl.ds(s * RPT + k * CH, CH), :])
    pltpu.sync_copy(rows_v.at[0].at[pl.ds(0, RPT % CH), :],
                    acc_sh.at[pl.ds(s * RPT + (RPT // CH) * CH, RPT % CH), :])
    plsc.subcore_barrier()

  scope = jax.named_scope("edge_main")
  scope.__enter__()
  stage(0, 0)
  g0 = issue_gather(0)
  stage(1, 1)
  issue_gather(1)
  g0.wait()
  scale(0)
  issue_scatter(0)

  npairs = jnp.where(c == 0, (NCH0 - 2) // 2, (NCH1 - 2) // 2)

  @pl.loop(0, npairs)
  def _pair(t):
    wait_scatter(0)
    stage(0, 2 * t + 2)
    issue_gather(0)
    wait_gather(1)
    scale(1)
    issue_scatter(1)
    wait_scatter(1)
    stage(1, 2 * t + 3)
    issue_gather(1)
    wait_gather(0)
    scale(0)
    issue_scatter(0)

  wait_gather(1)
  scale(1)
  issue_scatter(1)
  wait_scatter(0)
  wait_scatter(1)
  scope.__exit__(None, None, None)

  with jax.named_scope("acc_copyout"):
    plsc.subcore_barrier()
    nfull = RPT // CH
    sizes = [CH] * nfull + ([RPT % CH] if RPT % CH else [])
    for k, rows in enumerate(sizes):
      b = k % 2
      if k >= 2:
        pltpu.make_async_copy(
            rows_v.at[b].at[pl.ds(0, sizes[k - 2]), :],
            acc_hbm.at[c, pl.ds(s * RPT + (k - 2) * CH, sizes[k - 2]), :],
            sem_g[b]).wait()
      pltpu.sync_copy(acc_sh.at[pl.ds(s * RPT + k * CH, rows), :],
                      rows_v.at[b].at[pl.ds(0, rows), :])
      pltpu.async_copy(rows_v.at[b].at[pl.ds(0, rows), :],
                       acc_hbm.at[c, pl.ds(s * RPT + k * CH, rows), :],
                       sem_g[b])
    for k in (len(sizes) - 2, len(sizes) - 1):
      b = k % 2
      pltpu.make_async_copy(
          rows_v.at[b].at[pl.ds(0, sizes[k]), :],
          acc_hbm.at[c, pl.ds(s * RPT + k * CH, sizes[k]), :],
          sem_g[b]).wait()


def _make_sc_rows():
  mesh = plsc.VectorSubcoreMesh(core_axis_name="c", subcore_axis_name="s")
  return pl.kernel(
      _sc_rows_kernel,
      out_type=jax.ShapeDtypeStruct((NC, N, HP), jnp.float32),
      mesh=mesh,
      compiler_params=_SC_PARAMS,
      scratch_types=[
          pltpu.VMEM((2, 3, CH), jnp.int32),
          pltpu.VMEM((2, CH, HP), jnp.float32),
          pltpu.VMEM_SHARED((N, HP), jnp.float32),
          pltpu.SemaphoreType.DMA,
          pltpu.SemaphoreType.DMA,
          pltpu.SemaphoreType.DMA,
          pltpu.SemaphoreType.DMA,
      ],
  )


_HI = lax.Precision.HIGHEST


def _tc_head_body(x_ref, w_ref, ab_ref, h_ref, sc_ref):
  h = jnp.dot(x_ref[...], w_ref[...], preferred_element_type=jnp.float32,
              precision=_HI)
  rows = h.shape[0]
  h_ref[...] = jnp.concatenate(
      [h, jnp.ones((rows, HP - H), jnp.float32)], axis=1)
  sc_ref[...] = jnp.dot(h, ab_ref[...], preferred_element_type=jnp.float32,
                        precision=_HI)


def _tc_head(x, w, ab, rows_blk=1000):
  n = x.shape[0]
  grid = n // rows_blk
  return pl.pallas_call(
      _tc_head_body,
      grid=(grid,),
      in_specs=[
          pl.BlockSpec((rows_blk, D), lambda i: (i, 0)),
          pl.BlockSpec((D, H), lambda i: (0, 0)),
          pl.BlockSpec((H, 2), lambda i: (0, 0)),
      ],
      out_specs=[
          pl.BlockSpec((rows_blk, HP), lambda i: (i, 0)),
          pl.BlockSpec((rows_blk, 2), lambda i: (i, 0)),
      ],
      out_shape=[
          jax.ShapeDtypeStruct((n, HP), jnp.float32),
          jax.ShapeDtypeStruct((n, 2), jnp.float32),
      ],
  )(x, w, ab)


def _tc_combine_body(o_ref, b_ref, w_ref, ab_ref, gat_ref, h_ref, sc_ref):
  den = o_ref[0, :, H:H + 1] + o_ref[1, :, H:H + 1]
  g = (o_ref[0, :, :H] + o_ref[1, :, :H]) / (den + 1e-16) + b_ref[...]
  gat = jnp.maximum(g, 0.0)
  gat_ref[...] = gat
  h = jnp.dot(gat, w_ref[...], preferred_element_type=jnp.float32,
              precision=_HI)
  rows = h.shape[0]
  h_ref[...] = jnp.concatenate(
      [h, jnp.ones((rows, HP - H), jnp.float32)], axis=1)
  sc_ref[...] = jnp.dot(h, ab_ref[...], preferred_element_type=jnp.float32,
                        precision=_HI)


def _tc_combine(acc2, b, w, ab, rows_blk=1000):
  grid = N // rows_blk
  return pl.pallas_call(
      _tc_combine_body,
      grid=(grid,),
      in_specs=[
          pl.BlockSpec((NC, rows_blk, HP), lambda i: (0, i, 0)),
          pl.BlockSpec((1, H), lambda i: (0, 0)),
          pl.BlockSpec((H, H), lambda i: (0, 0)),
          pl.BlockSpec((H, 2), lambda i: (0, 0)),
      ],
      out_specs=[
          pl.BlockSpec((rows_blk, H), lambda i: (i, 0)),
          pl.BlockSpec((rows_blk, HP), lambda i: (i, 0)),
          pl.BlockSpec((rows_blk, 2), lambda i: (i, 0)),
      ],
      out_shape=[
          jax.ShapeDtypeStruct((N, H), jnp.float32),
          jax.ShapeDtypeStruct((N, HP), jnp.float32),
          jax.ShapeDtypeStruct((N, 2), jnp.float32),
      ],
  )(acc2, b, w, ab)


def _tc_final_body(o_ref, b_ref, batch_ref, wl_ref, bl_ref,
                   gat_ref, pooled_ref, final_ref, cnt_ref):
  i = pl.program_id(0)
  nsteps = pl.num_programs(0)
  den = o_ref[0, :, H:H + 1] + o_ref[1, :, H:H + 1]
  g = (o_ref[0, :, :H] + o_ref[1, :, :H]) / (den + 1e-16) + b_ref[...]
  gat_ref[...] = g

  rows = batch_ref.shape[2]
  gg = lax.broadcasted_iota(jnp.int32, (G, rows), 0)
  m = (gg == batch_ref[0]).astype(jnp.float32)

  @pl.when(i == 0)
  def _init():
    pooled_ref[...] = jnp.zeros_like(pooled_ref)
    cnt_ref[...] = jnp.zeros_like(cnt_ref)

  pooled_ref[...] += jnp.dot(m, g, preferred_element_type=jnp.float32,
                             precision=_HI)
  cnt_ref[...] += jnp.sum(m, axis=1, keepdims=True)

  @pl.when(i == nsteps - 1)
  def _fin():
    pooled = pooled_ref[...] / jnp.maximum(cnt_ref[...], 1.0)
    pooled_ref[...] = pooled
    final_ref[...] = jnp.dot(pooled, wl_ref[...],
                             preferred_element_type=jnp.float32,
                             precision=_HI) + bl_ref[...]


def _tc_final(acc2, b, batch3, wl, bl, rows_blk=1000):
  grid = N // rows_blk
  return pl.pallas_call(
      _tc_final_body,
      grid=(grid,),
      in_specs=[
          pl.BlockSpec((NC, rows_blk, HP), lambda i: (0, i, 0)),
          pl.BlockSpec((1, H), lambda i: (0, 0)),
          pl.BlockSpec((1, 1, rows_blk), lambda i: (i, 0, 0)),
          pl.BlockSpec((H, C), lambda i: (0, 0)),
          pl.BlockSpec((1, C), lambda i: (0, 0)),
      ],
      out_specs=[
          pl.BlockSpec((rows_blk, H), lambda i: (i, 0)),
          pl.BlockSpec((G, H), lambda i: (0, 0)),
          pl.BlockSpec((G, C), lambda i: (0, 0)),
      ],
      out_shape=[
          jax.ShapeDtypeStruct((N, H), jnp.float32),
          jax.ShapeDtypeStruct((G, H), jnp.float32),
          jax.ShapeDtypeStruct((G, C), jnp.float32),
      ],
      scratch_shapes=[pltpu.VMEM((G, 1), jnp.float32)],
  )(acc2, b, batch3, wl, bl)


def _per_tile_slabs(flat):
  c0 = flat[:NS * NCH0].reshape(NS, NCH0, CH)
  c1 = flat[NS * NCH0:].reshape(NS, NCH1, CH)
  c1 = jnp.pad(c1, ((0, 0), (0, NCH0 - NCH1), (0, 0)))
  return jnp.concatenate([c0, c1], axis=0)


def _from_tile_slabs(slabs):
  c0 = slabs[:NS].reshape(NS * NCH0, CH)
  c1 = slabs[NS:, :NCH1].reshape(NS * NCH1, CH)
  return jnp.concatenate([c0, c1], axis=0)


@jax.jit
def _forward(x, edge_index, batch,
             W1, a_src1, a_dst1, b1,
             W2, a_src2, a_dst2, b2,
             W3, a_src3, a_dst3, b3,
             W_lin, b_lin):
  pad = NCHT * CH - E
  srcF = jnp.pad(edge_index[0], (0, pad)).reshape(NCHT, CH)
  dstF = jnp.pad(edge_index[1], (0, pad)).reshape(NCHT, CH)
  srcP = _per_tile_slabs(srcF)
  dstP = _per_tile_slabs(dstF)
  batch3 = batch.reshape(N // 1000, 1, 1000)

  sc_pw = _make_sc_pw()
  sc_rows = _make_sc_rows()

  def edge_phase(h, sc):
    p3 = sc_pw(sc.T, srcP, dstP)
    pbits = lax.bitcast_convert_type(p3, jnp.int32)
    pF = _from_tile_slabs(pbits)
    sdp = jnp.stack([srcF, dstF, pF], axis=1)
    sdp = jnp.pad(sdp, ((0, 1), (0, 0), (0, 0)))
    return sc_rows(h, sdp)

  ab1 = jnp.stack([a_src1, a_dst1], axis=1)
  ab2 = jnp.stack([a_src2, a_dst2], axis=1)
  ab3 = jnp.stack([a_src3, a_dst3], axis=1)

  h1, sc1 = _tc_head(x, W1, ab1)
  acc1 = edge_phase(h1, sc1)
  gat1, h2, sc2 = _tc_combine(acc1, b1.reshape(1, H), W2, ab2)
  acc2 = edge_phase(h2, sc2)
  gat2, h3, sc3 = _tc_combine(acc2, b2.reshape(1, H), W3, ab3)
  acc3 = edge_phase(h3, sc3)
  gat3, pooled, final = _tc_final(acc3, b3.reshape(1, H), batch3,
                                  W_lin, b_lin.reshape(1, C))
  return gat1, gat2, gat3, pooled, final


def kernel(x, edge_index, batch, W1, a_src1, a_dst1, b1,
           W2, a_src2, a_dst2, b2, W3, a_src3, a_dst3, b3, W_lin, b_lin):
  return _forward(x, edge_index, batch,
                  W1, a_src1, a_dst1, b1,
                  W2, a_src2, a_dst2, b2,
                  W3, a_src3, a_dst3, b3,
                  W_lin, b_lin)


# speedup vs baseline: 1.4535x; 1.4535x over previous
import jax
import jax.numpy as jnp
from jax import lax
from jax.experimental import pallas as pl
from jax.experimental.pallas import tpu as pltpu
from jax.experimental.pallas import tpu_sc as plsc

N = 10000
E = 320000
D = 128
H = 128
HP = 144
C = 10
G = 64

NC = 2
NS = 16
NW = NC * NS
CH = 128
NCH0 = 80
NCH1 = 80
NCHT = NS * (NCH0 + NCH1)
RPT = N // NS

_SC_PARAMS = pltpu.CompilerParams(needs_layout_passes=False,
                                  use_tc_tiling_on_sc=False)


def _tile_plan(c, s):
  cbase = jnp.where(c == 0, s * NCH0, NS * NCH0 + s * NCH1)
  ncz = jnp.where(c == 0, NCH0, NCH1)
  return cbase, ncz


def _sc_pw_kernel(aa_hbm, src_hbm, dst_hbm, p_hbm,
                  asrc_v, adst_v, src2_v, dst2_v, p2_v):
  c = lax.axis_index("c")
  s = lax.axis_index("s")
  tid = c * NS + s
  cbase, ncz = _tile_plan(c, s)

  pltpu.sync_copy(aa_hbm.at[0], asrc_v)
  pltpu.sync_copy(aa_hbm.at[1], adst_v)
  pltpu.sync_copy(src_hbm.at[tid], src2_v)
  pltpu.sync_copy(dst_hbm.at[tid], dst2_v)

  @pl.loop(0, ncz)
  def _chunk(j):
    for i in range(CH // 16):
      s16 = src2_v[j, pl.ds(i * 16, 16)]
      d16 = dst2_v[j, pl.ds(i * 16, 16)]
      av = plsc.load_gather(asrc_v, [s16])
      dv = plsc.load_gather(adst_v, [d16])
      e = av + dv
      e = jnp.where(e < 0.0, 0.2 * e, e)
      p = jnp.exp(e)
      eid = (cbase + j) * CH + i * 16 + lax.iota(jnp.int32, 16)
      p = jnp.where(eid < E, p, 0.0)
      p2_v[j, pl.ds(i * 16, 16)] = p

  pltpu.sync_copy(p2_v, p_hbm.at[tid])


def _make_sc_pw():
  mesh = plsc.VectorSubcoreMesh(core_axis_name="c", subcore_axis_name="s")
  return pl.kernel(
      _sc_pw_kernel,
      out_type=jax.ShapeDtypeStruct((NW, NCH0, CH), jnp.float32),
      mesh=mesh,
      compiler_params=_SC_PARAMS,
      scratch_types=[
          pltpu.VMEM((N,), jnp.float32),
          pltpu.VMEM((N,), jnp.float32),
          pltpu.VMEM((NCH0, CH), jnp.int32),
          pltpu.VMEM((NCH0, CH), jnp.int32),
          pltpu.VMEM((NCH0, CH), jnp.float32),
      ],
  )


def _sc_rows_kernel(h_hbm, sdp_hbm,
                    acc_hbm,
                    sdp_v, rows_v, acc_sh,
                    sem_g0, sem_g1, sem_s0, sem_s1):
  c = lax.axis_index("c")
  s = lax.axis_index("s")
  cbase, ncz = _tile_plan(c, s)
  sem_g = (sem_g0, sem_g1)
  sem_s = (sem_s0, sem_s1)

  def stage(b, j):
    pltpu.sync_copy(sdp_hbm.at[cbase + j], sdp_v.at[b])

  def issue_gather(b):
    return pltpu.async_copy(h_hbm.at[sdp_v.at[b, 0]], rows_v.at[b],
                            sem_g[b])

  def wait_gather(b):
    pltpu.make_async_copy(h_hbm.at[sdp_v.at[b, 0]], rows_v.at[b],
                          sem_g[b]).wait()

  def issue_scatter(b):
    return pltpu.async_copy(rows_v.at[b], acc_sh.at[sdp_v.at[b, 1]],
                            sem_s[b], add=True)

  def wait_scatter(b):
    pltpu.make_async_copy(rows_v.at[b], acc_sh.at[sdp_v.at[b, 1]],
                          sem_s[b]).wait()

  def scale(b):
    for k in range(CH // 16):
      p16 = plsc.bitcast(sdp_v[b, 2, pl.ds(k * 16, 16)], jnp.float32)
      for t in range(16):
        r = k * 16 + t
        pr = p16[t]
        for i in range(HP // 16):
          rows_v[b, r, pl.ds(i * 16, 16)] = (
              rows_v[b, r, pl.ds(i * 16, 16)] * pr)

  with jax.named_scope("acc_zero"):
    @pl.loop(0, CH)
    def _zero_rows(r):
      for i in range(HP // 16):
        rows_v[0, r, pl.ds(i * 16, 16)] = jnp.zeros((16,), jnp.float32)

    for k in range(RPT // CH):
      pltpu.sync_copy(rows_v.at[0],
                      acc_sh.at[pl.ds(s * RPT + k * CH, CH), :])
    pltpu.sync_copy(rows_v.at[0].at[pl.ds(0, RPT % CH), :],
                    acc_sh.at[pl.ds(s * RPT + (RPT // CH) * CH, RPT % CH), :])
    plsc.subcore_barrier()

  scope = jax.named_scope("edge_main")
  scope.__enter__()
  stage(0, 0)
  g0 = issue_gather(0)
  stage(1, 1)
  issue_gather(1)
  g0.wait()
  scale(0)
  issue_scatter(0)

  npairs = jnp.where(c == 0, (NCH0 - 2) // 2, (NCH1 - 2) // 2)

  @pl.loop(0, npairs)
  def _pair(t):
    wait_scatter(0)
    stage(0, 2 * t + 2)
    issue_gather(0)
    wait_gather(1)
    scale(1)
    issue_scatter(1)
    wait_scatter(1)
    stage(1, 2 * t + 3)
    issue_gather(1)
    wait_gather(0)
    scale(0)
    issue_scatter(0)

  wait_gather(1)
  scale(1)
  issue_scatter(1)
  wait_scatter(0)
  wait_scatter(1)
  scope.__exit__(None, None, None)

  with jax.named_scope("acc_copyout"):
    plsc.subcore_barrier()
    nfull = RPT // CH
    sizes = [CH] * nfull + ([RPT % CH] if RPT % CH else [])
    for k, rows in enumerate(sizes):
      b = k % 2
      if k >= 2:
        pltpu.make_async_copy(
            rows_v.at[b].at[pl.ds(0, sizes[k - 2]), :],
            acc_hbm.at[c, pl.ds(s * RPT + (k - 2) * CH, sizes[k - 2]), :],
            sem_g[b]).wait()
      pltpu.sync_copy(acc_sh.at[pl.ds(s * RPT + k * CH, rows), :],
                      rows_v.at[b].at[pl.ds(0, rows), :])
      pltpu.async_copy(rows_v.at[b].at[pl.ds(0, rows), :],
                       acc_hbm.at[c, pl.ds(s * RPT + k * CH, rows), :],
                       sem_g[b])
    for k in (len(sizes) - 2, len(sizes) - 1):
      b = k % 2
      pltpu.make_async_copy(
          rows_v.at[b].at[pl.ds(0, sizes[k]), :],
          acc_hbm.at[c, pl.ds(s * RPT + k * CH, sizes[k]), :],
          sem_g[b]).wait()


def _make_sc_rows():
  mesh = plsc.VectorSubcoreMesh(core_axis_name="c", subcore_axis_name="s")
  return pl.kernel(
      _sc_rows_kernel,
      out_type=jax.ShapeDtypeStruct((NC, N, HP), jnp.float32),
      mesh=mesh,
      compiler_params=_SC_PARAMS,
      scratch_types=[
          pltpu.VMEM((2, 3, CH), jnp.int32),
          pltpu.VMEM((2, CH, HP), jnp.float32),
          pltpu.VMEM_SHARED((N, HP), jnp.float32),
          pltpu.SemaphoreType.DMA,
          pltpu.SemaphoreType.DMA,
          pltpu.SemaphoreType.DMA,
          pltpu.SemaphoreType.DMA,
      ],
  )


_HI = lax.Precision.HIGHEST


def _tc_head_body(x_ref, w_ref, ab_ref, h_ref, sc_ref):
  h = jnp.dot(x_ref[...], w_ref[...], preferred_element_type=jnp.float32,
              precision=_HI)
  rows = h.shape[0]
  h_ref[...] = jnp.concatenate(
      [h, jnp.ones((rows, HP - H), jnp.float32)], axis=1)
  sc_ref[...] = jnp.dot(h, ab_ref[...], preferred_element_type=jnp.float32,
                        precision=_HI)


def _tc_head(x, w, ab, rows_blk=1000):
  n = x.shape[0]
  grid = n // rows_blk
  return pl.pallas_call(
      _tc_head_body,
      grid=(grid,),
      in_specs=[
          pl.BlockSpec((rows_blk, D), lambda i: (i, 0)),
          pl.BlockSpec((D, H), lambda i: (0, 0)),
          pl.BlockSpec((H, 2), lambda i: (0, 0)),
      ],
      out_specs=[
          pl.BlockSpec((rows_blk, HP), lambda i: (i, 0)),
          pl.BlockSpec((rows_blk, 2), lambda i: (i, 0)),
      ],
      out_shape=[
          jax.ShapeDtypeStruct((n, HP), jnp.float32),
          jax.ShapeDtypeStruct((n, 2), jnp.float32),
      ],
  )(x, w, ab)


def _tc_combine_body(o_ref, b_ref, w_ref, ab_ref, gat_ref, h_ref, sc_ref):
  den = o_ref[0, :, H:H + 1] + o_ref[1, :, H:H + 1]
  g = (o_ref[0, :, :H] + o_ref[1, :, :H]) / (den + 1e-16) + b_ref[...]
  gat = jnp.maximum(g, 0.0)
  gat_ref[...] = gat
  h = jnp.dot(gat, w_ref[...], preferred_element_type=jnp.float32,
              precision=_HI)
  rows = h.shape[0]
  h_ref[...] = jnp.concatenate(
      [h, jnp.ones((rows, HP - H), jnp.float32)], axis=1)
  sc_ref[...] = jnp.dot(h, ab_ref[...], preferred_element_type=jnp.float32,
                        precision=_HI)


def _tc_combine(acc2, b, w, ab, rows_blk=1000):
  grid = N // rows_blk
  return pl.pallas_call(
      _tc_combine_body,
      grid=(grid,),
      in_specs=[
          pl.BlockSpec((NC, rows_blk, HP), lambda i: (0, i, 0)),
          pl.BlockSpec((1, H), lambda i: (0, 0)),
          pl.BlockSpec((H, H), lambda i: (0, 0)),
          pl.BlockSpec((H, 2), lambda i: (0, 0)),
      ],
      out_specs=[
          pl.BlockSpec((rows_blk, H), lambda i: (i, 0)),
          pl.BlockSpec((rows_blk, HP), lambda i: (i, 0)),
          pl.BlockSpec((rows_blk, 2), lambda i: (i, 0)),
      ],
      out_shape=[
          jax.ShapeDtypeStruct((N, H), jnp.float32),
          jax.ShapeDtypeStruct((N, HP), jnp.float32),
          jax.ShapeDtypeStruct((N, 2), jnp.float32),
      ],
  )(acc2, b, w, ab)


def _tc_final_body(o_ref, b_ref, batch_ref, wl_ref, bl_ref,
                   gat_ref, pooled_ref, final_ref, cnt_ref):
  i = pl.program_id(0)
  nsteps = pl.num_programs(0)
  den = o_ref[0, :, H:H + 1] + o_ref[1, :, H:H + 1]
  g = (o_ref[0, :, :H] + o_ref[1, :, :H]) / (den + 1e-16) + b_ref[...]
  gat_ref[...] = g

  rows = batch_ref.shape[2]
  gg = lax.broadcasted_iota(jnp.int32, (G, rows), 0)
  m = (gg == batch_ref[0]).astype(jnp.float32)

  @pl.when(i == 0)
  def _init():
    pooled_ref[...] = jnp.zeros_like(pooled_ref)
    cnt_ref[...] = jnp.zeros_like(cnt_ref)

  pooled_ref[...] += jnp.dot(m, g, preferred_element_type=jnp.float32,
                             precision=_HI)
  cnt_ref[...] += jnp.sum(m, axis=1, keepdims=True)

  @pl.when(i == nsteps - 1)
  def _fin():
    pooled = pooled_ref[...] / jnp.maximum(cnt_ref[...], 1.0)
    pooled_ref[...] = pooled
    final_ref[...] = jnp.dot(pooled, wl_ref[...],
                             preferred_element_type=jnp.float32,
                             precision=_HI) + bl_ref[...]


def _tc_final(acc2, b, batch3, wl, bl, rows_blk=1000):
  grid = N // rows_blk
  return pl.pallas_call(
      _tc_final_body,
      grid=(grid,),
      in_specs=[
          pl.BlockSpec((NC, rows_blk, HP), lambda i: (0, i, 0)),
          pl.BlockSpec((1, H), lambda i: (0, 0)),
          pl.BlockSpec((1, 1, rows_blk), lambda i: (i, 0, 0)),
          pl.BlockSpec((H, C), lambda i: (0, 0)),
          pl.BlockSpec((1, C), lambda i: (0, 0)),
      ],
      out_specs=[
          pl.BlockSpec((rows_blk, H), lambda i: (i, 0)),
          pl.BlockSpec((G, H), lambda i: (0, 0)),
          pl.BlockSpec((G, C), lambda i: (0, 0)),
      ],
      out_shape=[
          jax.ShapeDtypeStruct((N, H), jnp.float32),
          jax.ShapeDtypeStruct((G, H), jnp.float32),
          jax.ShapeDtypeStruct((G, C), jnp.float32),
      ],
      scratch_shapes=[pltpu.VMEM((G, 1), jnp.float32)],
  )(acc2, b, batch3, wl, bl)


def _per_tile_slabs(flat):
  c0 = flat[:NS * NCH0].reshape(NS, NCH0, CH)
  c1 = flat[NS * NCH0:].reshape(NS, NCH1, CH)
  c1 = jnp.pad(c1, ((0, 0), (0, NCH0 - NCH1), (0, 0)))
  return jnp.concatenate([c0, c1], axis=0)


def _from_tile_slabs(slabs):
  c0 = slabs[:NS].reshape(NS * NCH0, CH)
  c1 = slabs[NS:, :NCH1].reshape(NS * NCH1, CH)
  return jnp.concatenate([c0, c1], axis=0)


@jax.jit
def _forward(x, edge_index, batch,
             W1, a_src1, a_dst1, b1,
             W2, a_src2, a_dst2, b2,
             W3, a_src3, a_dst3, b3,
             W_lin, b_lin):
  pad = NCHT * CH - E
  pad_idx = jnp.arange(pad, dtype=jnp.int32) % N
  srcF = jnp.concatenate([edge_index[0], pad_idx]).reshape(NCHT, CH)
  dstF = jnp.concatenate([edge_index[1], pad_idx]).reshape(NCHT, CH)
  srcP = _per_tile_slabs(srcF)
  dstP = _per_tile_slabs(dstF)
  batch3 = batch.reshape(N // 1000, 1, 1000)

  sc_pw = _make_sc_pw()
  sc_rows = _make_sc_rows()

  def edge_phase(h, sc):
    p3 = sc_pw(sc.T, srcP, dstP)
    pbits = lax.bitcast_convert_type(p3, jnp.int32)
    pF = _from_tile_slabs(pbits)
    sdp = jnp.stack([srcF, dstF, pF], axis=1)
    sdp = jnp.pad(sdp, ((0, 1), (0, 0), (0, 0)))
    return sc_rows(h, sdp)

  ab1 = jnp.stack([a_src1, a_dst1], axis=1)
  ab2 = jnp.stack([a_src2, a_dst2], axis=1)
  ab3 = jnp.stack([a_src3, a_dst3], axis=1)

  h1, sc1 = _tc_head(x, W1, ab1)
  acc1 = edge_phase(h1, sc1)
  gat1, h2, sc2 = _tc_combine(acc1, b1.reshape(1, H), W2, ab2)
  acc2 = edge_phase(h2, sc2)
  gat2, h3, sc3 = _tc_combine(acc2, b2.reshape(1, H), W3, ab3)
  acc3 = edge_phase(h3, sc3)
  gat3, pooled, final = _tc_final(acc3, b3.reshape(1, H), batch3,
                                  W_lin, b_lin.reshape(1, C))
  return gat1, gat2, gat3, pooled, final


def kernel(x, edge_index, batch, W1, a_src1, a_dst1, b1,
           W2, a_src2, a_dst2, b2, W3, a_src3, a_dst3, b3, W_lin, b_lin):
  return _forward(x, edge_index, batch,
                  W1, a_src1, a_dst1, b1,
                  W2, a_src2, a_dst2, b2,
                  W3, a_src3, a_dst3, b3,
                  W_lin, b_lin)


# speedup vs baseline: 2.1337x; 1.4680x over previous
import jax
import jax.numpy as jnp
from jax import lax
from jax.experimental import pallas as pl
from jax.experimental.pallas import tpu as pltpu
from jax.experimental.pallas import tpu_sc as plsc

N = 10000
E = 320000
D = 128
H = 128
HP = 144
C = 10
G = 64

NC = 2
NS = 16
NW = NC * NS
CH = 128
NCH0 = 80
NCH1 = 80
NCHT = NS * (NCH0 + NCH1)
RPT = N // NS

_SC_PARAMS = pltpu.CompilerParams(needs_layout_passes=False,
                                  use_tc_tiling_on_sc=False)


def _tile_plan(c, s):
  cbase = jnp.where(c == 0, s * NCH0, NS * NCH0 + s * NCH1)
  ncz = jnp.where(c == 0, NCH0, NCH1)
  return cbase, ncz


def _sc_pw_kernel(aa_hbm, src_hbm, dst_hbm, p_hbm,
                  asrc_v, adst_v, src2_v, dst2_v, p2_v):
  c = lax.axis_index("c")
  s = lax.axis_index("s")
  tid = c * NS + s
  cbase, ncz = _tile_plan(c, s)

  pltpu.sync_copy(aa_hbm.at[0], asrc_v)
  pltpu.sync_copy(aa_hbm.at[1], adst_v)
  pltpu.sync_copy(src_hbm.at[tid], src2_v)
  pltpu.sync_copy(dst_hbm.at[tid], dst2_v)

  @pl.loop(0, ncz)
  def _chunk(j):
    for i in range(CH // 16):
      s16 = src2_v[j, pl.ds(i * 16, 16)]
      d16 = dst2_v[j, pl.ds(i * 16, 16)]
      av = plsc.load_gather(asrc_v, [s16])
      dv = plsc.load_gather(adst_v, [d16])
      e = av + dv
      e = jnp.where(e < 0.0, 0.2 * e, e)
      p = jnp.exp(e)
      eid = (cbase + j) * CH + i * 16 + lax.iota(jnp.int32, 16)
      p = jnp.where(eid < E, p, 0.0)
      p2_v[j, pl.ds(i * 16, 16)] = p

  pltpu.sync_copy(p2_v, p_hbm.at[tid])


def _make_sc_pw():
  mesh = plsc.VectorSubcoreMesh(core_axis_name="c", subcore_axis_name="s")
  return pl.kernel(
      _sc_pw_kernel,
      out_type=jax.ShapeDtypeStruct((NW, NCH0, CH), jnp.float32),
      mesh=mesh,
      compiler_params=_SC_PARAMS,
      scratch_types=[
          pltpu.VMEM((N,), jnp.float32),
          pltpu.VMEM((N,), jnp.float32),
          pltpu.VMEM((NCH0, CH), jnp.int32),
          pltpu.VMEM((NCH0, CH), jnp.int32),
          pltpu.VMEM((NCH0, CH), jnp.float32),
      ],
  )


def _sc_rows_kernel(h_hbm, sdp_hbm,
                    acc_hbm,
                    sdp_v, rows_v, acc_sh,
                    sem_g0, sem_g1, sem_s0, sem_s1,
                    sem_t0, sem_t1, sem_t2, sem_t3):
  c = lax.axis_index("c")
  s = lax.axis_index("s")
  cbase, ncz = _tile_plan(c, s)
  sem_g = (sem_g0, sem_g1)
  sem_s = (sem_s0, sem_s1)
  sem_t = (sem_t0, sem_t1, sem_t2, sem_t3)

  def stage_async(sb, j):
    pltpu.async_copy(sdp_hbm.at[cbase + j], sdp_v.at[sb], sem_t[sb])

  def wait_stage(sb, j):
    pltpu.make_async_copy(sdp_hbm.at[cbase + j], sdp_v.at[sb],
                          sem_t[sb]).wait()

  def issue_gather(b, sb):
    return pltpu.async_copy(h_hbm.at[sdp_v.at[sb, 0]], rows_v.at[b],
                            sem_g[b])

  def wait_gather(b, sb):
    pltpu.make_async_copy(h_hbm.at[sdp_v.at[sb, 0]], rows_v.at[b],
                          sem_g[b]).wait()

  def issue_scatter(b, sb):
    return pltpu.async_copy(rows_v.at[b], acc_sh.at[sdp_v.at[sb, 1]],
                            sem_s[b], add=True)

  def wait_scatter(b, sb):
    pltpu.make_async_copy(rows_v.at[b], acc_sh.at[sdp_v.at[sb, 1]],
                          sem_s[b]).wait()

  def scale(b, sb):
    @pl.loop(0, CH // 16)
    def _group(k):
      p16 = plsc.bitcast(sdp_v[sb, 2, pl.ds(k * 16, 16)], jnp.float32)
      for t in range(16):
        r = k * 16 + t
        pr = p16[t]
        for i in range(HP // 16):
          rows_v[b, r, pl.ds(i * 16, 16)] = (
              rows_v[b, r, pl.ds(i * 16, 16)] * pr)

  with jax.named_scope("acc_zero"):
    @pl.loop(0, CH)
    def _zero_rows(r):
      for i in range(HP // 16):
        rows_v[0, r, pl.ds(i * 16, 16)] = jnp.zeros((16,), jnp.float32)

    for k in range(RPT // CH):
      pltpu.sync_copy(rows_v.at[0],
                      acc_sh.at[pl.ds(s * RPT + k * CH, CH), :])
    pltpu.sync_copy(rows_v.at[0].at[pl.ds(0, RPT % CH), :],
                    acc_sh.at[pl.ds(s * RPT + (RPT // CH) * CH, RPT % CH), :])
    plsc.subcore_barrier()

  scope = jax.named_scope("edge_main")
  scope.__enter__()
  NCH = NCH0

  stage_async(0, 0)
  stage_async(1, 1)
  stage_async(2, 2)
  stage_async(3, 3)
  wait_stage(0, 0)
  issue_gather(0, 0)
  wait_stage(1, 1)
  issue_gather(1, 1)
  wait_gather(0, 0)
  scale(0, 0)
  issue_scatter(0, 0)
  wait_scatter(0, 0)
  wait_stage(2, 2)
  issue_gather(0, 2)
  wait_gather(1, 1)
  scale(1, 1)
  issue_scatter(1, 1)

  @pl.loop(0, (NCH - 2) // 4)
  def _quad(t):
    for q in range(4):
      j = 4 * t + 2 + q
      sb = (2 + q) % 4
      b = q % 2
      stage_async((sb + 2) % 4, j + 2)
      wait_scatter(1 - b, (sb + 3) % 4)
      wait_stage((sb + 1) % 4, j + 1)
      issue_gather(1 - b, (sb + 1) % 4)
      wait_gather(b, sb)
      scale(b, sb)
      issue_scatter(b, sb)

  wait_scatter(1, 1)
  wait_stage(3, NCH - 1)
  issue_gather(1, 3)
  wait_gather(0, 2)
  scale(0, 2)
  issue_scatter(0, 2)
  wait_gather(1, 3)
  scale(1, 3)
  issue_scatter(1, 3)
  wait_scatter(0, 2)
  wait_scatter(1, 3)
  scope.__exit__(None, None, None)

  with jax.named_scope("acc_copyout"):
    plsc.subcore_barrier()
    nfull = RPT // CH
    sizes = [CH] * nfull + ([RPT % CH] if RPT % CH else [])
    for k, rows in enumerate(sizes):
      b = k % 2
      if k >= 2:
        pltpu.make_async_copy(
            rows_v.at[b].at[pl.ds(0, sizes[k - 2]), :],
            acc_hbm.at[c, pl.ds(s * RPT + (k - 2) * CH, sizes[k - 2]), :],
            sem_g[b]).wait()
      pltpu.sync_copy(acc_sh.at[pl.ds(s * RPT + k * CH, rows), :],
                      rows_v.at[b].at[pl.ds(0, rows), :])
      pltpu.async_copy(rows_v.at[b].at[pl.ds(0, rows), :],
                       acc_hbm.at[c, pl.ds(s * RPT + k * CH, rows), :],
                       sem_g[b])
    for k in (len(sizes) - 2, len(sizes) - 1):
      b = k % 2
      pltpu.make_async_copy(
          rows_v.at[b].at[pl.ds(0, sizes[k]), :],
          acc_hbm.at[c, pl.ds(s * RPT + k * CH, sizes[k]), :],
          sem_g[b]).wait()


def _make_sc_rows():
  mesh = plsc.VectorSubcoreMesh(core_axis_name="c", subcore_axis_name="s")
  return pl.kernel(
      _sc_rows_kernel,
      out_type=jax.ShapeDtypeStruct((NC, N, HP), jnp.float32),
      mesh=mesh,
      compiler_params=_SC_PARAMS,
      scratch_types=[
          pltpu.VMEM((4, 3, CH), jnp.int32),
          pltpu.VMEM((2, CH, HP), jnp.float32),
          pltpu.VMEM_SHARED((N, HP), jnp.float32),
          pltpu.SemaphoreType.DMA,
          pltpu.SemaphoreType.DMA,
          pltpu.SemaphoreType.DMA,
          pltpu.SemaphoreType.DMA,
          pltpu.SemaphoreType.DMA,
          pltpu.SemaphoreType.DMA,
          pltpu.SemaphoreType.DMA,
          pltpu.SemaphoreType.DMA,
      ],
  )


_HI = lax.Precision.HIGHEST


def _tc_head_body(x_ref, w_ref, ab_ref, h_ref, sc_ref):
  h = jnp.dot(x_ref[...], w_ref[...], preferred_element_type=jnp.float32,
              precision=_HI)
  rows = h.shape[0]
  h_ref[...] = jnp.concatenate(
      [h, jnp.ones((rows, HP - H), jnp.float32)], axis=1)
  sc_ref[...] = jnp.dot(h, ab_ref[...], preferred_element_type=jnp.float32,
                        precision=_HI)


def _tc_head(x, w, ab, rows_blk=1000):
  n = x.shape[0]
  grid = n // rows_blk
  return pl.pallas_call(
      _tc_head_body,
      grid=(grid,),
      in_specs=[
          pl.BlockSpec((rows_blk, D), lambda i: (i, 0)),
          pl.BlockSpec((D, H), lambda i: (0, 0)),
          pl.BlockSpec((H, 2), lambda i: (0, 0)),
      ],
      out_specs=[
          pl.BlockSpec((rows_blk, HP), lambda i: (i, 0)),
          pl.BlockSpec((rows_blk, 2), lambda i: (i, 0)),
      ],
      out_shape=[
          jax.ShapeDtypeStruct((n, HP), jnp.float32),
          jax.ShapeDtypeStruct((n, 2), jnp.float32),
      ],
  )(x, w, ab)


def _tc_combine_body(o_ref, b_ref, w_ref, ab_ref, gat_ref, h_ref, sc_ref):
  den = o_ref[0, :, H:H + 1] + o_ref[1, :, H:H + 1]
  g = (o_ref[0, :, :H] + o_ref[1, :, :H]) / (den + 1e-16) + b_ref[...]
  gat = jnp.maximum(g, 0.0)
  gat_ref[...] = gat
  h = jnp.dot(gat, w_ref[...], preferred_element_type=jnp.float32,
              precision=_HI)
  rows = h.shape[0]
  h_ref[...] = jnp.concatenate(
      [h, jnp.ones((rows, HP - H), jnp.float32)], axis=1)
  sc_ref[...] = jnp.dot(h, ab_ref[...], preferred_element_type=jnp.float32,
                        precision=_HI)


def _tc_combine(acc2, b, w, ab, rows_blk=1000):
  grid = N // rows_blk
  return pl.pallas_call(
      _tc_combine_body,
      grid=(grid,),
      in_specs=[
          pl.BlockSpec((NC, rows_blk, HP), lambda i: (0, i, 0)),
          pl.BlockSpec((1, H), lambda i: (0, 0)),
          pl.BlockSpec((H, H), lambda i: (0, 0)),
          pl.BlockSpec((H, 2), lambda i: (0, 0)),
      ],
      out_specs=[
          pl.BlockSpec((rows_blk, H), lambda i: (i, 0)),
          pl.BlockSpec((rows_blk, HP), lambda i: (i, 0)),
          pl.BlockSpec((rows_blk, 2), lambda i: (i, 0)),
      ],
      out_shape=[
          jax.ShapeDtypeStruct((N, H), jnp.float32),
          jax.ShapeDtypeStruct((N, HP), jnp.float32),
          jax.ShapeDtypeStruct((N, 2), jnp.float32),
      ],
  )(acc2, b, w, ab)


def _tc_final_body(o_ref, b_ref, batch_ref, wl_ref, bl_ref,
                   gat_ref, pooled_ref, final_ref, cnt_ref):
  i = pl.program_id(0)
  nsteps = pl.num_programs(0)
  den = o_ref[0, :, H:H + 1] + o_ref[1, :, H:H + 1]
  g = (o_ref[0, :, :H] + o_ref[1, :, :H]) / (den + 1e-16) + b_ref[...]
  gat_ref[...] = g

  rows = batch_ref.shape[2]
  gg = lax.broadcasted_iota(jnp.int32, (G, rows), 0)
  m = (gg == batch_ref[0]).astype(jnp.float32)

  @pl.when(i == 0)
  def _init():
    pooled_ref[...] = jnp.zeros_like(pooled_ref)
    cnt_ref[...] = jnp.zeros_like(cnt_ref)

  pooled_ref[...] += jnp.dot(m, g, preferred_element_type=jnp.float32,
                             precision=_HI)
  cnt_ref[...] += jnp.sum(m, axis=1, keepdims=True)

  @pl.when(i == nsteps - 1)
  def _fin():
    pooled = pooled_ref[...] / jnp.maximum(cnt_ref[...], 1.0)
    pooled_ref[...] = pooled
    final_ref[...] = jnp.dot(pooled, wl_ref[...],
                             preferred_element_type=jnp.float32,
                             precision=_HI) + bl_ref[...]


def _tc_final(acc2, b, batch3, wl, bl, rows_blk=1000):
  grid = N // rows_blk
  return pl.pallas_call(
      _tc_final_body,
      grid=(grid,),
      in_specs=[
          pl.BlockSpec((NC, rows_blk, HP), lambda i: (0, i, 0)),
          pl.BlockSpec((1, H), lambda i: (0, 0)),
          pl.BlockSpec((1, 1, rows_blk), lambda i: (i, 0, 0)),
          pl.BlockSpec((H, C), lambda i: (0, 0)),
          pl.BlockSpec((1, C), lambda i: (0, 0)),
      ],
      out_specs=[
          pl.BlockSpec((rows_blk, H), lambda i: (i, 0)),
          pl.BlockSpec((G, H), lambda i: (0, 0)),
          pl.BlockSpec((G, C), lambda i: (0, 0)),
      ],
      out_shape=[
          jax.ShapeDtypeStruct((N, H), jnp.float32),
          jax.ShapeDtypeStruct((G, H), jnp.float32),
          jax.ShapeDtypeStruct((G, C), jnp.float32),
      ],
      scratch_shapes=[pltpu.VMEM((G, 1), jnp.float32)],
  )(acc2, b, batch3, wl, bl)


def _per_tile_slabs(flat):
  c0 = flat[:NS * NCH0].reshape(NS, NCH0, CH)
  c1 = flat[NS * NCH0:].reshape(NS, NCH1, CH)
  c1 = jnp.pad(c1, ((0, 0), (0, NCH0 - NCH1), (0, 0)))
  return jnp.concatenate([c0, c1], axis=0)


def _from_tile_slabs(slabs):
  c0 = slabs[:NS].reshape(NS * NCH0, CH)
  c1 = slabs[NS:, :NCH1].reshape(NS * NCH1, CH)
  return jnp.concatenate([c0, c1], axis=0)


@jax.jit
def _forward(x, edge_index, batch,
             W1, a_src1, a_dst1, b1,
             W2, a_src2, a_dst2, b2,
             W3, a_src3, a_dst3, b3,
             W_lin, b_lin):
  pad = NCHT * CH - E
  pad_idx = jnp.arange(pad, dtype=jnp.int32) % N
  srcF = jnp.concatenate([edge_index[0], pad_idx]).reshape(NCHT, CH)
  dstF = jnp.concatenate([edge_index[1], pad_idx]).reshape(NCHT, CH)
  srcP = _per_tile_slabs(srcF)
  dstP = _per_tile_slabs(dstF)
  batch3 = batch.reshape(N // 1000, 1, 1000)

  sc_pw = _make_sc_pw()
  sc_rows = _make_sc_rows()

  def edge_phase(h, sc):
    p3 = sc_pw(sc.T, srcP, dstP)
    pbits = lax.bitcast_convert_type(p3, jnp.int32)
    pF = _from_tile_slabs(pbits)
    sdp = jnp.stack([srcF, dstF, pF], axis=1)
    sdp = jnp.pad(sdp, ((0, 1), (0, 0), (0, 0)))
    return sc_rows(h, sdp)

  ab1 = jnp.stack([a_src1, a_dst1], axis=1)
  ab2 = jnp.stack([a_src2, a_dst2], axis=1)
  ab3 = jnp.stack([a_src3, a_dst3], axis=1)

  h1, sc1 = _tc_head(x, W1, ab1)
  acc1 = edge_phase(h1, sc1)
  gat1, h2, sc2 = _tc_combine(acc1, b1.reshape(1, H), W2, ab2)
  acc2 = edge_phase(h2, sc2)
  gat2, h3, sc3 = _tc_combine(acc2, b2.reshape(1, H), W3, ab3)
  acc3 = edge_phase(h3, sc3)
  gat3, pooled, final = _tc_final(acc3, b3.reshape(1, H), batch3,
                                  W_lin, b_lin.reshape(1, C))
  return gat1, gat2, gat3, pooled, final


def kernel(x, edge_index, batch, W1, a_src1, a_dst1, b1,
           W2, a_src2, a_dst2, b2, W3, a_src3, a_dst3, b3, W_lin, b_lin):
  return _forward(x, edge_index, batch,
                  W1, a_src1, a_dst1, b1,
                  W2, a_src2, a_dst2, b2,
                  W3, a_src3, a_dst3, b3,
                  W_lin, b_lin)


# speedup vs baseline: 2.3046x; 1.0801x over previous
import jax
import jax.numpy as jnp
from jax import lax
from jax.experimental import pallas as pl
from jax.experimental.pallas import tpu as pltpu
from jax.experimental.pallas import tpu_sc as plsc

N = 10000
E = 320000
D = 128
H = 128
HP = 144
C = 10
G = 64

NC = 2
NS = 16
NW = NC * NS
CH = 128
NCH0 = 80
NCH1 = 80
NCHT = NS * (NCH0 + NCH1)
RPT = N // NS

_SC_PARAMS = pltpu.CompilerParams(needs_layout_passes=False,
                                  use_tc_tiling_on_sc=False)


def _tile_plan(c, s):
  cbase = jnp.where(c == 0, s * NCH0, NS * NCH0 + s * NCH1)
  ncz = jnp.where(c == 0, NCH0, NCH1)
  return cbase, ncz


def _sc_pw_kernel(aa_hbm, src_hbm, dst_hbm, p_hbm,
                  asrc_v, adst_v, src2_v, dst2_v, p2_v):
  c = lax.axis_index("c")
  s = lax.axis_index("s")
  tid = c * NS + s
  cbase, ncz = _tile_plan(c, s)

  pltpu.sync_copy(aa_hbm.at[0], asrc_v)
  pltpu.sync_copy(aa_hbm.at[1], adst_v)
  pltpu.sync_copy(src_hbm.at[tid], src2_v)
  pltpu.sync_copy(dst_hbm.at[tid], dst2_v)

  @pl.loop(0, ncz)
  def _chunk(j):
    for i in range(CH // 16):
      s16 = src2_v[j, pl.ds(i * 16, 16)]
      d16 = dst2_v[j, pl.ds(i * 16, 16)]
      av = plsc.load_gather(asrc_v, [s16])
      dv = plsc.load_gather(adst_v, [d16])
      e = av + dv
      e = jnp.where(e < 0.0, 0.2 * e, e)
      p = jnp.exp(e)
      eid = (cbase + j) * CH + i * 16 + lax.iota(jnp.int32, 16)
      p = jnp.where(eid < E, p, 0.0)
      p2_v[j, pl.ds(i * 16, 16)] = p

  pltpu.sync_copy(p2_v, p_hbm.at[tid])


def _make_sc_pw():
  mesh = plsc.VectorSubcoreMesh(core_axis_name="c", subcore_axis_name="s")
  return pl.kernel(
      _sc_pw_kernel,
      out_type=jax.ShapeDtypeStruct((NW, NCH0, CH), jnp.float32),
      mesh=mesh,
      compiler_params=_SC_PARAMS,
      scratch_types=[
          pltpu.VMEM((N,), jnp.float32),
          pltpu.VMEM((N,), jnp.float32),
          pltpu.VMEM((NCH0, CH), jnp.int32),
          pltpu.VMEM((NCH0, CH), jnp.int32),
          pltpu.VMEM((NCH0, CH), jnp.float32),
      ],
  )


def _sc_rows_kernel(h_hbm, sdp_hbm,
                    acc_hbm,
                    sdp_v, rows_v, acc_sh,
                    sem_g0, sem_g1, sem_s0, sem_s1,
                    sem_t0, sem_t1, sem_t2, sem_t3):
  c = lax.axis_index("c")
  s = lax.axis_index("s")
  cbase, ncz = _tile_plan(c, s)
  sem_g = (sem_g0, sem_g1)
  sem_s = (sem_s0, sem_s1)
  sem_t = (sem_t0, sem_t1, sem_t2, sem_t3)

  def stage_async(sb, j):
    pltpu.async_copy(sdp_hbm.at[cbase + j], sdp_v.at[sb], sem_t[sb])

  def wait_stage(sb, j):
    pltpu.make_async_copy(sdp_hbm.at[cbase + j], sdp_v.at[sb],
                          sem_t[sb]).wait()

  def issue_gather(b, sb):
    return pltpu.async_copy(h_hbm.at[sdp_v.at[sb, 0]], rows_v.at[b],
                            sem_g[b])

  def wait_gather(b, sb):
    pltpu.make_async_copy(h_hbm.at[sdp_v.at[sb, 0]], rows_v.at[b],
                          sem_g[b]).wait()

  def issue_scatter(b, sb):
    return pltpu.async_copy(rows_v.at[b], acc_sh.at[sdp_v.at[sb, 1]],
                            sem_s[b], add=True)

  def wait_scatter(b, sb):
    pltpu.make_async_copy(rows_v.at[b], acc_sh.at[sdp_v.at[sb, 1]],
                          sem_s[b]).wait()

  def scale(b, sb):
    @pl.loop(0, CH // 16)
    def _group(k):
      p16 = plsc.bitcast(sdp_v[sb, 2, pl.ds(k * 16, 16)], jnp.float32)
      for t in range(16):
        r = k * 16 + t
        pr = p16[t]
        for i in range(HP // 16):
          rows_v[b, r, pl.ds(i * 16, 16)] = (
              rows_v[b, r, pl.ds(i * 16, 16)] * pr)

  with jax.named_scope("acc_zero"):
    @pl.loop(0, CH)
    def _zero_rows(r):
      for i in range(HP // 16):
        rows_v[0, r, pl.ds(i * 16, 16)] = jnp.zeros((16,), jnp.float32)

    for k in range(RPT // CH):
      pltpu.sync_copy(rows_v.at[0],
                      acc_sh.at[pl.ds(s * RPT + k * CH, CH), :])
    pltpu.sync_copy(rows_v.at[0].at[pl.ds(0, RPT % CH), :],
                    acc_sh.at[pl.ds(s * RPT + (RPT // CH) * CH, RPT % CH), :])
    plsc.subcore_barrier()

  scope = jax.named_scope("edge_main")
  scope.__enter__()
  NCH = NCH0

  stage_async(0, 0)
  stage_async(1, 1)
  stage_async(2, 2)
  stage_async(3, 3)
  wait_stage(0, 0)
  issue_gather(0, 0)
  wait_stage(1, 1)
  issue_gather(1, 1)
  wait_gather(0, 0)
  scale(0, 0)
  issue_scatter(0, 0)
  wait_scatter(0, 0)
  wait_stage(2, 2)
  issue_gather(0, 2)
  wait_gather(1, 1)
  scale(1, 1)
  issue_scatter(1, 1)

  @pl.loop(0, (NCH - 2) // 4)
  def _quad(t):
    for q in range(4):
      j = 4 * t + 2 + q
      sb = (2 + q) % 4
      b = q % 2
      stage_async((sb + 2) % 4, j + 2)
      wait_scatter(1 - b, (sb + 3) % 4)
      wait_stage((sb + 1) % 4, j + 1)
      issue_gather(1 - b, (sb + 1) % 4)
      wait_gather(b, sb)
      scale(b, sb)
      issue_scatter(b, sb)

  wait_scatter(1, 1)
  wait_stage(3, NCH - 1)
  issue_gather(1, 3)
  wait_gather(0, 2)
  scale(0, 2)
  issue_scatter(0, 2)
  wait_gather(1, 3)
  scale(1, 3)
  issue_scatter(1, 3)
  wait_scatter(0, 2)
  wait_scatter(1, 3)
  scope.__exit__(None, None, None)

  with jax.named_scope("acc_copyout"):
    plsc.subcore_barrier()
    nfull = RPT // CH
    sizes = [CH] * nfull + ([RPT % CH] if RPT % CH else [])
    for k, rows in enumerate(sizes):
      b = k % 2
      if k >= 2:
        pltpu.make_async_copy(
            rows_v.at[b].at[pl.ds(0, sizes[k - 2]), :],
            acc_hbm.at[c, pl.ds(s * RPT + (k - 2) * CH, sizes[k - 2]), :],
            sem_g[b]).wait()
      pltpu.sync_copy(acc_sh.at[pl.ds(s * RPT + k * CH, rows), :],
                      rows_v.at[b].at[pl.ds(0, rows), :])
      pltpu.async_copy(rows_v.at[b].at[pl.ds(0, rows), :],
                       acc_hbm.at[c, pl.ds(s * RPT + k * CH, rows), :],
                       sem_g[b])
    for k in (len(sizes) - 2, len(sizes) - 1):
      b = k % 2
      pltpu.make_async_copy(
          rows_v.at[b].at[pl.ds(0, sizes[k]), :],
          acc_hbm.at[c, pl.ds(s * RPT + k * CH, sizes[k]), :],
          sem_g[b]).wait()


def _make_sc_rows():
  mesh = plsc.VectorSubcoreMesh(core_axis_name="c", subcore_axis_name="s")
  return pl.kernel(
      _sc_rows_kernel,
      out_type=jax.ShapeDtypeStruct((NC, N, HP), jnp.float32),
      mesh=mesh,
      compiler_params=_SC_PARAMS,
      scratch_types=[
          pltpu.VMEM((4, 3, CH), jnp.int32),
          pltpu.VMEM((2, CH, HP), jnp.float32),
          pltpu.VMEM_SHARED((N, HP), jnp.float32),
          pltpu.SemaphoreType.DMA,
          pltpu.SemaphoreType.DMA,
          pltpu.SemaphoreType.DMA,
          pltpu.SemaphoreType.DMA,
          pltpu.SemaphoreType.DMA,
          pltpu.SemaphoreType.DMA,
          pltpu.SemaphoreType.DMA,
          pltpu.SemaphoreType.DMA,
      ],
  )


_HI = lax.Precision.DEFAULT


def _tc_head_body(x_ref, w_ref, ab_ref, h_ref, sc_ref):
  h = jnp.dot(x_ref[...], w_ref[...], preferred_element_type=jnp.float32,
              precision=_HI)
  rows = h.shape[0]
  h_ref[...] = jnp.concatenate(
      [h, jnp.ones((rows, HP - H), jnp.float32)], axis=1)
  sc_ref[...] = jnp.dot(h, ab_ref[...], preferred_element_type=jnp.float32,
                        precision=_HI)


def _tc_head(x, w, ab, rows_blk=1000):
  n = x.shape[0]
  grid = n // rows_blk
  return pl.pallas_call(
      _tc_head_body,
      grid=(grid,),
      in_specs=[
          pl.BlockSpec((rows_blk, D), lambda i: (i, 0)),
          pl.BlockSpec((D, H), lambda i: (0, 0)),
          pl.BlockSpec((H, 2), lambda i: (0, 0)),
      ],
      out_specs=[
          pl.BlockSpec((rows_blk, HP), lambda i: (i, 0)),
          pl.BlockSpec((rows_blk, 2), lambda i: (i, 0)),
      ],
      out_shape=[
          jax.ShapeDtypeStruct((n, HP), jnp.float32),
          jax.ShapeDtypeStruct((n, 2), jnp.float32),
      ],
  )(x, w, ab)


def _tc_combine_body(o_ref, b_ref, w_ref, ab_ref, gat_ref, h_ref, sc_ref):
  den = o_ref[0, :, H:H + 1] + o_ref[1, :, H:H + 1]
  g = (o_ref[0, :, :H] + o_ref[1, :, :H]) / (den + 1e-16) + b_ref[...]
  gat = jnp.maximum(g, 0.0)
  gat_ref[...] = gat
  h = jnp.dot(gat, w_ref[...], preferred_element_type=jnp.float32,
              precision=_HI)
  rows = h.shape[0]
  h_ref[...] = jnp.concatenate(
      [h, jnp.ones((rows, HP - H), jnp.float32)], axis=1)
  sc_ref[...] = jnp.dot(h, ab_ref[...], preferred_element_type=jnp.float32,
                        precision=_HI)


def _tc_combine(acc2, b, w, ab, rows_blk=1000):
  grid = N // rows_blk
  return pl.pallas_call(
      _tc_combine_body,
      grid=(grid,),
      in_specs=[
          pl.BlockSpec((NC, rows_blk, HP), lambda i: (0, i, 0)),
          pl.BlockSpec((1, H), lambda i: (0, 0)),
          pl.BlockSpec((H, H), lambda i: (0, 0)),
          pl.BlockSpec((H, 2), lambda i: (0, 0)),
      ],
      out_specs=[
          pl.BlockSpec((rows_blk, H), lambda i: (i, 0)),
          pl.BlockSpec((rows_blk, HP), lambda i: (i, 0)),
          pl.BlockSpec((rows_blk, 2), lambda i: (i, 0)),
      ],
      out_shape=[
          jax.ShapeDtypeStruct((N, H), jnp.float32),
          jax.ShapeDtypeStruct((N, HP), jnp.float32),
          jax.ShapeDtypeStruct((N, 2), jnp.float32),
      ],
  )(acc2, b, w, ab)


def _tc_final_body(o_ref, b_ref, batch_ref, wl_ref, bl_ref,
                   gat_ref, pooled_ref, final_ref, cnt_ref):
  i = pl.program_id(0)
  nsteps = pl.num_programs(0)
  den = o_ref[0, :, H:H + 1] + o_ref[1, :, H:H + 1]
  g = (o_ref[0, :, :H] + o_ref[1, :, :H]) / (den + 1e-16) + b_ref[...]
  gat_ref[...] = g

  rows = batch_ref.shape[2]
  gg = lax.broadcasted_iota(jnp.int32, (G, rows), 0)
  m = (gg == batch_ref[0]).astype(jnp.float32)

  @pl.when(i == 0)
  def _init():
    pooled_ref[...] = jnp.zeros_like(pooled_ref)
    cnt_ref[...] = jnp.zeros_like(cnt_ref)

  pooled_ref[...] += jnp.dot(m, g, preferred_element_type=jnp.float32,
                             precision=_HI)
  cnt_ref[...] += jnp.sum(m, axis=1, keepdims=True)

  @pl.when(i == nsteps - 1)
  def _fin():
    pooled = pooled_ref[...] / jnp.maximum(cnt_ref[...], 1.0)
    pooled_ref[...] = pooled
    final_ref[...] = jnp.dot(pooled, wl_ref[...],
                             preferred_element_type=jnp.float32,
                             precision=_HI) + bl_ref[...]


def _tc_final(acc2, b, batch3, wl, bl, rows_blk=1000):
  grid = N // rows_blk
  return pl.pallas_call(
      _tc_final_body,
      grid=(grid,),
      in_specs=[
          pl.BlockSpec((NC, rows_blk, HP), lambda i: (0, i, 0)),
          pl.BlockSpec((1, H), lambda i: (0, 0)),
          pl.BlockSpec((1, 1, rows_blk), lambda i: (i, 0, 0)),
          pl.BlockSpec((H, C), lambda i: (0, 0)),
          pl.BlockSpec((1, C), lambda i: (0, 0)),
      ],
      out_specs=[
          pl.BlockSpec((rows_blk, H), lambda i: (i, 0)),
          pl.BlockSpec((G, H), lambda i: (0, 0)),
          pl.BlockSpec((G, C), lambda i: (0, 0)),
      ],
      out_shape=[
          jax.ShapeDtypeStruct((N, H), jnp.float32),
          jax.ShapeDtypeStruct((G, H), jnp.float32),
          jax.ShapeDtypeStruct((G, C), jnp.float32),
      ],
      scratch_shapes=[pltpu.VMEM((G, 1), jnp.float32)],
  )(acc2, b, batch3, wl, bl)


def _per_tile_slabs(flat):
  c0 = flat[:NS * NCH0].reshape(NS, NCH0, CH)
  c1 = flat[NS * NCH0:].reshape(NS, NCH1, CH)
  c1 = jnp.pad(c1, ((0, 0), (0, NCH0 - NCH1), (0, 0)))
  return jnp.concatenate([c0, c1], axis=0)


def _from_tile_slabs(slabs):
  c0 = slabs[:NS].reshape(NS * NCH0, CH)
  c1 = slabs[NS:, :NCH1].reshape(NS * NCH1, CH)
  return jnp.concatenate([c0, c1], axis=0)


@jax.jit
def _forward(x, edge_index, batch,
             W1, a_src1, a_dst1, b1,
             W2, a_src2, a_dst2, b2,
             W3, a_src3, a_dst3, b3,
             W_lin, b_lin):
  pad = NCHT * CH - E
  pad_idx = jnp.arange(pad, dtype=jnp.int32) % N
  srcF = jnp.concatenate([edge_index[0], pad_idx]).reshape(NCHT, CH)
  dstF = jnp.concatenate([edge_index[1], pad_idx]).reshape(NCHT, CH)
  srcP = _per_tile_slabs(srcF)
  dstP = _per_tile_slabs(dstF)
  batch3 = batch.reshape(N // 1000, 1, 1000)

  sc_pw = _make_sc_pw()
  sc_rows = _make_sc_rows()

  def edge_phase(h, sc):
    p3 = sc_pw(sc.T, srcP, dstP)
    pbits = lax.bitcast_convert_type(p3, jnp.int32)
    pF = _from_tile_slabs(pbits)
    sdp = jnp.stack([srcF, dstF, pF], axis=1)
    sdp = jnp.pad(sdp, ((0, 1), (0, 0), (0, 0)))
    return sc_rows(h, sdp)

  ab1 = jnp.stack([a_src1, a_dst1], axis=1)
  ab2 = jnp.stack([a_src2, a_dst2], axis=1)
  ab3 = jnp.stack([a_src3, a_dst3], axis=1)

  h1, sc1 = _tc_head(x, W1, ab1)
  acc1 = edge_phase(h1, sc1)
  gat1, h2, sc2 = _tc_combine(acc1, b1.reshape(1, H), W2, ab2)
  acc2 = edge_phase(h2, sc2)
  gat2, h3, sc3 = _tc_combine(acc2, b2.reshape(1, H), W3, ab3)
  acc3 = edge_phase(h3, sc3)
  gat3, pooled, final = _tc_final(acc3, b3.reshape(1, H), batch3,
                                  W_lin, b_lin.reshape(1, C))
  return gat1, gat2, gat3, pooled, final


def kernel(x, edge_index, batch, W1, a_src1, a_dst1, b1,
           W2, a_src2, a_dst2, b2, W3, a_src3, a_dst3, b3, W_lin, b_lin):
  return _forward(x, edge_index, batch,
                  W1, a_src1, a_dst1, b1,
                  W2, a_src2, a_dst2, b2,
                  W3, a_src3, a_dst3, b3,
                  W_lin, b_lin)
